# Initial kernel scaffold; baseline (speedup 1.0000x reference)
import jax
import jax.numpy as jnp
from jax.experimental import pallas as pl


def kernel(x, edge_index, pseudo, W1, root1, bias1, W2, root2, bias2):
    raise NotImplementedError("write your pallas kernel here")


# speedup vs baseline: 19.1223x; 19.1223x over previous
import functools

import jax
import jax.numpy as jnp
from jax import lax
from jax.experimental import pallas as pl
from jax.experimental.pallas import tpu as pltpu
from jax.experimental.pallas import tpu_sc as plsc

N = 50000
E = 800000
NT = 32
NO = 8
NQ = 4
ES = E // NQ
C = 2000
NCHUNK = ES // C
EB = 6400
NB = 3125


def _phase_a_body(pt_ref, mt_ref, bt_ref):
    pt = pt_ref[...]
    v = pt * 4.0
    fl = jnp.floor(v)
    frac = v - fl
    k0 = jnp.clip(fl.astype(jnp.int32), 0, 3)
    io5 = lax.broadcasted_iota(jnp.int32, (5, EB), 0)

    def cdim(d):
        k = k0[d:d + 1]
        f = frac[d:d + 1]
        return jnp.where(io5 == k, 1.0 - f, jnp.where(io5 == k + 1, f, 0.0))

    c0 = cdim(0)
    c1 = cdim(1)
    c2 = cdim(2)
    c12 = (c2[:, None, :] * c1[None, :, :]).reshape(25, EB)
    dt = jax.lax.dot_general(mt_ref[...], c12, (((1,), (0,)), ((), ())),
                             preferred_element_type=jnp.float32)
    acc = dt[0:16] * c0[0:1]
    for i in range(1, 5):
        acc = acc + dt[i * 16:(i + 1) * 16] * c0[i:i + 1]
    bt_ref[...] = acc


def _phase_a(pseudo_t, mt):
    return pl.pallas_call(
        _phase_a_body,
        grid=(E // EB,),
        in_specs=[pl.BlockSpec((3, EB), lambda i: (0, i)),
                  pl.BlockSpec((80, 25), lambda i: (0, 0))],
        out_specs=pl.BlockSpec((16, EB), lambda i: (0, i)),
        out_shape=jax.ShapeDtypeStruct((16, E), jnp.float32),
    )(pseudo_t, mt)


def _make_sc_layer(table_rows, brow_offset):
    mesh = plsc.VectorSubcoreMesh(core_axis_name="c", subcore_axis_name="s")

    @functools.partial(
        pl.kernel,
        mesh=mesh,
        compiler_params=pltpu.CompilerParams(needs_layout_passes=False),
        out_type=jax.ShapeDtypeStruct((NT * N,), jnp.float32),
        scratch_types=[
            pltpu.VMEM((N,), jnp.float32),
            pltpu.VMEM((N,), jnp.float32),
            pltpu.VMEM((C,), jnp.int32),
            pltpu.VMEM((C,), jnp.int32),
            pltpu.VMEM((C,), jnp.float32),
        ],
    )
    def sc_layer(table_hbm, src_hbm, dst_hbm, bt_hbm, out_hbm,
                 tab_v, agg_v, src_v, dst_v, b_v):
        wid = lax.axis_index("s") * 2 + lax.axis_index("c")
        o = wid // NQ
        q = wid % NQ
        trow = o if table_rows == NO else 0
        pltpu.sync_copy(table_hbm.at[pl.ds(pl.multiple_of(trow * N, 8), N)],
                        tab_v)
        zeros16 = jnp.zeros((16,), jnp.float32)

        def zbody(i, carry):
            agg_v[pl.ds(i * 16, 16)] = zeros16
            return carry

        lax.fori_loop(0, N // 16, zbody, 0)

        ebase = q * ES
        brow = brow_offset + o

        def chunk(ci, carry):
            off = pl.multiple_of(ebase + ci * C, 8)
            boff = pl.multiple_of(brow * E + ebase + ci * C, 8)
            pltpu.sync_copy(src_hbm.at[pl.ds(off, C)], src_v)
            pltpu.sync_copy(dst_hbm.at[pl.ds(off, C)], dst_v)
            pltpu.sync_copy(bt_hbm.at[pl.ds(boff, C)], b_v)

            def inner(j, icarry):
                sl = pl.ds(j * 16, 16)
                si = src_v[sl]
                xs = plsc.load_gather(tab_v, [si])
                msg = xs * b_v[sl]
                di = dst_v[sl]
                plsc.addupdate_scatter(agg_v, [di], msg)
                return icarry

            lax.fori_loop(0, C // 16, inner, 0)
            return carry

        lax.fori_loop(0, NCHUNK, chunk, 0)
        pltpu.sync_copy(agg_v, out_hbm.at[pl.ds(pl.multiple_of(wid * N, 8), N)])

    return sc_layer


_sc_layer1 = _make_sc_layer(1, 0)
_sc_layer2 = _make_sc_layer(NO, NO)


def _phase_c_body(p_ref, x_ref, r_ref, b_ref, h_ref):
    s = p_ref[...].reshape(NO, NQ, N).sum(axis=1)
    h = s + x_ref[...] * r_ref[...] + b_ref[...]
    h_ref[...] = jnp.where(h > 0, h, jnp.exp(jnp.minimum(h, 0.0)) - 1.0)


def _phase_c(p1, x_row, root1_c, bias1_c):
    return pl.pallas_call(
        _phase_c_body,
        out_shape=jax.ShapeDtypeStruct((NO, N), jnp.float32),
    )(p1, x_row, root1_c, bias1_c)


def _phase_d_body(p_ref, h_ref, r_ref, b_ref, o_ref):
    s = p_ref[...].sum(axis=0, keepdims=True)
    hr = (h_ref[...] * r_ref[...]).sum(axis=0, keepdims=True)
    v = s + hr + b_ref[...]
    o_ref[...] = jnp.where(v > 0, v, jnp.exp(jnp.minimum(v, 0.0)) - 1.0)


def _phase_d(p2, ht, root2_c, bias2_c):
    return pl.pallas_call(
        _phase_d_body,
        out_shape=jax.ShapeDtypeStruct((1, N), jnp.float32),
    )(p2, ht, root2_c, bias2_c)


def kernel(x, edge_index, pseudo, W1, root1, bias1, W2, root2, bias2):
    src = edge_index[0].astype(jnp.int32)
    dst = edge_index[1].astype(jnp.int32)
    pseudo_t = pseudo.T
    wcat = jnp.concatenate([W1[:, 0, :], W2[:, :, 0]], axis=1)
    mt = wcat.reshape(5, 5, 5, 16).reshape(25, 80).T

    bt = _phase_a(pseudo_t, mt)
    btf = bt.reshape(-1)
    xf = x.reshape(-1)

    p1 = _sc_layer1(xf, src, dst, btf)
    ht = _phase_c(p1.reshape(NT, N), x.reshape(1, N),
                  root1.reshape(NO, 1), bias1.reshape(NO, 1))

    p2 = _sc_layer2(ht.reshape(-1), src, dst, btf)
    out = _phase_d(p2.reshape(NT, N), ht,
                   root2.reshape(NO, 1), bias2.reshape(1, 1))
    return out.reshape(-1)


# speedup vs baseline: 36.2879x; 1.8977x over previous
import functools

import jax
import jax.numpy as jnp
from jax import lax
from jax.experimental import pallas as pl
from jax.experimental.pallas import tpu as pltpu
from jax.experimental.pallas import tpu_sc as plsc

N = 50000
E = 800000
NT = 32
NO = 8
NQ = 4
ES = E // NQ
C = 2000
NCHUNK = ES // C
E_PAD = 819200
EB = 8192

SEGC = 12512
SEGD = 1664

_SC_PARAMS = pltpu.CompilerParams(needs_layout_passes=False)


def _elu16(h):
    return jnp.where(h > 0, h, jnp.exp(jnp.minimum(h, 0.0)) - 1.0)


def _phase_a_body(pt_ref, mt_ref, *bt_refs):
    pt = pt_ref[...]
    v = pt * 4.0
    fl = jnp.floor(v)
    frac = v - fl
    k0 = jnp.clip(fl.astype(jnp.int32), 0, 3)
    io5 = lax.broadcasted_iota(jnp.int32, (5, EB), 0)

    def cdim(d):
        k = k0[d:d + 1]
        f = frac[d:d + 1]
        return jnp.where(io5 == k, 1.0 - f, jnp.where(io5 == k + 1, f, 0.0))

    c0 = cdim(0)
    c1 = cdim(1)
    c2 = cdim(2)
    c12 = (c2[:, None, :] * c1[None, :, :]).reshape(25, EB)
    dt = jax.lax.dot_general(mt_ref[...], c12, (((1,), (0,)), ((), ())),
                             preferred_element_type=jnp.float32)
    acc = dt[0:16] * c0[0:1]
    for i in range(1, 5):
        acc = acc + dt[i * 16:(i + 1) * 16] * c0[i:i + 1]
    for r in range(16):
        bt_refs[r][...] = acc[r]


def _phase_a(pseudo_t, mt):
    return pl.pallas_call(
        _phase_a_body,
        grid=(E_PAD // EB,),
        in_specs=[pl.BlockSpec((3, EB), lambda i: (0, i)),
                  pl.BlockSpec((80, 25), lambda i: (0, 0))],
        out_specs=[pl.BlockSpec((EB,), lambda i: (i,)) for _ in range(16)],
        out_shape=[jax.ShapeDtypeStruct((E_PAD,), jnp.float32)
                   for _ in range(16)],
    )(pseudo_t, mt)


def _make_sc_layer(table_rows):
    mesh = plsc.VectorSubcoreMesh(core_axis_name="c", subcore_axis_name="s")

    @functools.partial(
        pl.kernel,
        mesh=mesh,
        compiler_params=_SC_PARAMS,
        out_type=jax.ShapeDtypeStruct((NT * N,), jnp.float32),
        scratch_types=[
            pltpu.VMEM((N,), jnp.float32),
            pltpu.VMEM((N,), jnp.float32),
            pltpu.VMEM((C,), jnp.int32),
            pltpu.VMEM((C,), jnp.int32),
            pltpu.VMEM((C,), jnp.float32),
        ],
    )
    def sc_layer(table_hbm, src_hbm, dst_hbm, b0, b1, b2, b3, b4, b5, b6, b7,
                 out_hbm, tab_v, agg_v, src_v, dst_v, b_v):
        brows = (b0, b1, b2, b3, b4, b5, b6, b7)
        wid = lax.axis_index("s") * 2 + lax.axis_index("c")
        o = wid % NO
        q = wid // NO
        if table_rows == NO:
            pltpu.sync_copy(table_hbm.at[pl.ds(o * N, N)], tab_v)
        else:
            pltpu.sync_copy(table_hbm, tab_v)
        zeros16 = jnp.zeros((16,), jnp.float32)

        def zbody(i, carry):
            agg_v[pl.ds(i * 16, 16)] = zeros16
            return carry

        lax.fori_loop(0, N // 16, zbody, 0)

        ebase = q * ES

        def chunk(ci, carry):
            off = pl.multiple_of(ebase + ci * C, 8)
            pltpu.sync_copy(src_hbm.at[pl.ds(off, C)], src_v)
            pltpu.sync_copy(dst_hbm.at[pl.ds(off, C)], dst_v)
            for r in range(NO):
                @pl.when(o == r)
                def _():
                    pltpu.sync_copy(brows[r].at[pl.ds(off, C)], b_v)

            def inner(j, icarry):
                sl = pl.ds(j * 16, 16)
                si = src_v[sl]
                xs = plsc.load_gather(tab_v, [si])
                msg = xs * b_v[sl]
                di = dst_v[sl]
                plsc.addupdate_scatter(agg_v, [di], msg)
                return icarry

            lax.fori_loop(0, C // 16, inner, 0)
            return carry

        lax.fori_loop(0, NCHUNK, chunk, 0)
        pltpu.sync_copy(agg_v, out_hbm.at[pl.ds(wid * N, N)])

    return sc_layer


_sc_layer1 = _make_sc_layer(1)
_sc_layer2 = _make_sc_layer(NO)


def _make_sc_combine1():
    mesh = plsc.VectorSubcoreMesh(core_axis_name="c", subcore_axis_name="s")

    @functools.partial(
        pl.kernel,
        mesh=mesh,
        compiler_params=_SC_PARAMS,
        out_type=jax.ShapeDtypeStruct((NO * N,), jnp.float32),
        scratch_types=[
            pltpu.VMEM((SEGC,), jnp.float32),
            pltpu.VMEM((SEGC,), jnp.float32),
            pltpu.VMEM((16,), jnp.float32),
            pltpu.VMEM((16,), jnp.float32),
        ],
    )
    def sc_c(p_hbm, x_hbm, rb_hbm, bb_hbm, out_hbm, acc_v, stg_v, r_v, b_v):
        wid = lax.axis_index("s") * 2 + lax.axis_index("c")
        o = wid % NO
        q = wid // NO
        start = pl.multiple_of(
            jnp.where(q == NQ - 1, N - SEGC, q * SEGC).astype(jnp.int32), 16)
        pltpu.sync_copy(rb_hbm.at[pl.ds(o * 16, 16)], r_v)
        pltpu.sync_copy(bb_hbm.at[pl.ds(o * 16, 16)], b_v)
        pltpu.sync_copy(p_hbm.at[pl.ds(o * N + start, SEGC)], acc_v)
        for j in range(1, NQ):
            pltpu.sync_copy(p_hbm.at[pl.ds((j * NO + o) * N + start, SEGC)],
                            stg_v)

            def abody(k, carry, _j=j):
                sl = pl.ds(k * 16, 16)
                acc_v[sl] = acc_v[sl] + stg_v[sl]
                return carry

            lax.fori_loop(0, SEGC // 16, abody, 0)
        pltpu.sync_copy(x_hbm.at[pl.ds(start, SEGC)], stg_v)
        rv = r_v[...]
        bv = b_v[...]

        def fbody(k, carry):
            sl = pl.ds(k * 16, 16)
            h = acc_v[sl] + stg_v[sl] * rv + bv
            acc_v[sl] = _elu16(h)
            return carry

        lax.fori_loop(0, SEGC // 16, fbody, 0)
        pltpu.sync_copy(acc_v, out_hbm.at[pl.ds(o * N + start, SEGC)])

    return sc_c


_sc_combine1 = _make_sc_combine1()


def _make_sc_combine2():
    mesh = plsc.VectorSubcoreMesh(core_axis_name="c", subcore_axis_name="s")

    @functools.partial(
        pl.kernel,
        mesh=mesh,
        compiler_params=_SC_PARAMS,
        out_type=jax.ShapeDtypeStruct((N,), jnp.float32),
        scratch_types=[
            pltpu.VMEM(((NT + NO) * SEGD,), jnp.float32),
            pltpu.VMEM((SEGD,), jnp.float32),
            pltpu.VMEM((128,), jnp.float32),
            pltpu.VMEM((16,), jnp.float32),
            pltpu.SemaphoreType.DMA,
        ],
    )
    def sc_d(p_hbm, h_hbm, rb_hbm, bb_hbm, out_hbm, buf_v, acc_v, r_v, b_v,
             sem):
        wid = lax.axis_index("s") * 2 + lax.axis_index("c")
        start = pl.multiple_of(
            jnp.minimum(wid * SEGD, N - SEGD).astype(jnp.int32), 16)
        pltpu.sync_copy(rb_hbm, r_v)
        pltpu.sync_copy(bb_hbm, b_v)
        cps = []
        for j in range(NT):
            cps.append(pltpu.async_copy(
                p_hbm.at[pl.ds(j * N + start, SEGD)],
                buf_v.at[pl.ds(j * SEGD, SEGD)], sem))
        for o2 in range(NO):
            cps.append(pltpu.async_copy(
                h_hbm.at[pl.ds(o2 * N + start, SEGD)],
                buf_v.at[pl.ds((NT + o2) * SEGD, SEGD)], sem))
        for cp in cps:
            cp.wait()
        rv = [r_v[pl.ds(o2 * 16, 16)] for o2 in range(NO)]
        bv = b_v[...]

        def fbody(k, carry):
            s = buf_v[pl.ds(k * 16, 16)]
            for j in range(1, NT):
                s = s + buf_v[pl.ds(j * SEGD + k * 16, 16)]
            for o2 in range(NO):
                s = s + buf_v[pl.ds((NT + o2) * SEGD + k * 16, 16)] * rv[o2]
            acc_v[pl.ds(k * 16, 16)] = _elu16(s + bv)
            return carry

        lax.fori_loop(0, SEGD // 16, fbody, 0)
        pltpu.sync_copy(acc_v, out_hbm.at[pl.ds(start, SEGD)])

    return sc_d


_sc_combine2 = _make_sc_combine2()


def kernel(x, edge_index, pseudo, W1, root1, bias1, W2, root2, bias2):
    src = edge_index[0].astype(jnp.int32)
    dst = edge_index[1].astype(jnp.int32)
    pseudo_t = jnp.pad(pseudo.T, ((0, 0), (0, E_PAD - E)))
    wcat = jnp.concatenate([W1[:, 0, :], W2[:, :, 0]], axis=1)
    mt = wcat.reshape(5, 5, 5, 16).reshape(25, 80).T
    xf = x.reshape(-1)
    rb1 = jnp.tile(root1.reshape(NO, 1), (1, 16)).reshape(-1)
    bb1 = jnp.tile(bias1.reshape(NO, 1), (1, 16)).reshape(-1)
    rb2 = jnp.tile(root2.reshape(NO, 1), (1, 16)).reshape(-1)
    bb2 = jnp.broadcast_to(bias2, (16,)).astype(jnp.float32)

    bt = _phase_a(pseudo_t, mt)
    p1 = _sc_layer1(xf, src, dst, *bt[0:8])
    htf = _sc_combine1(p1, xf, rb1, bb1)
    p2 = _sc_layer2(htf, src, dst, *bt[8:16])
    return _sc_combine2(p2, htf, rb2, bb2)


# speedup vs baseline: 65.8322x; 1.8142x over previous
import functools

import jax
import jax.numpy as jnp
from jax import lax
from jax.experimental import pallas as pl
from jax.experimental.pallas import tpu as pltpu
from jax.experimental.pallas import tpu_sc as plsc

N = 50000
E = 800000
NT = 32
NO = 8
NQ = 4
ES = E // NQ
C = 2000
NCHUNK = ES // C
NPAIR = NCHUNK // 2
E_PAD = 819200
EB = 8192

SEGC = 12544
SEGD = 1664

_SC_PARAMS = pltpu.CompilerParams(needs_layout_passes=False)


def _elu16(h):
    return jnp.where(h > 0, h, jnp.exp(jnp.minimum(h, 0.0)) - 1.0)


def _phase_a_body(pt_ref, mt_ref, *bt_refs):
    pt = pt_ref[...]
    v = pt * 4.0
    fl = jnp.floor(v)
    frac = v - fl
    k0 = jnp.clip(fl.astype(jnp.int32), 0, 3)
    io5 = lax.broadcasted_iota(jnp.int32, (5, EB), 0)

    def cdim(d):
        k = k0[d:d + 1]
        f = frac[d:d + 1]
        return jnp.where(io5 == k, 1.0 - f, jnp.where(io5 == k + 1, f, 0.0))

    c0 = cdim(0)
    c1 = cdim(1)
    c2 = cdim(2)
    c12 = (c2[:, None, :] * c1[None, :, :]).reshape(25, EB)
    dt = jax.lax.dot_general(mt_ref[...], c12, (((1,), (0,)), ((), ())),
                             preferred_element_type=jnp.float32)
    acc = dt[0:16] * c0[0:1]
    for i in range(1, 5):
        acc = acc + dt[i * 16:(i + 1) * 16] * c0[i:i + 1]
    for r in range(16):
        bt_refs[r][...] = acc[r]


def _phase_a(pseudo_t, mt):
    return pl.pallas_call(
        _phase_a_body,
        grid=(E_PAD // EB,),
        in_specs=[pl.BlockSpec((3, EB), lambda i: (0, i)),
                  pl.BlockSpec((80, 25), lambda i: (0, 0))],
        out_specs=[pl.BlockSpec((EB,), lambda i: (i,)) for _ in range(16)],
        out_shape=[jax.ShapeDtypeStruct((E_PAD,), jnp.float32)
                   for _ in range(16)],
    )(pseudo_t, mt)


def _make_sc_layer(table_rows):
    mesh = plsc.VectorSubcoreMesh(core_axis_name="c", subcore_axis_name="s")

    @functools.partial(
        pl.kernel,
        mesh=mesh,
        compiler_params=_SC_PARAMS,
        out_type=jax.ShapeDtypeStruct((NT * N,), jnp.float32),
        scratch_types=[
            pltpu.VMEM((N,), jnp.float32),
            pltpu.VMEM((N,), jnp.float32),
            pltpu.VMEM((C,), jnp.int32),
            pltpu.VMEM((C,), jnp.int32),
            pltpu.VMEM((C,), jnp.float32),
            pltpu.VMEM((C,), jnp.int32),
            pltpu.VMEM((C,), jnp.int32),
            pltpu.VMEM((C,), jnp.float32),
            pltpu.SemaphoreType.DMA,
            pltpu.SemaphoreType.DMA,
        ],
    )
    def sc_layer(table_hbm, src_hbm, dst_hbm, b0, b1, b2, b3, b4, b5, b6, b7,
                 out_hbm, tab_v, agg_v, sA, dA, bA, sB, dB, bB, semA, semB):
        brows = (b0, b1, b2, b3, b4, b5, b6, b7)
        wid = lax.axis_index("s") * 2 + lax.axis_index("c")
        o = wid % NO
        q = wid // NO
        if table_rows == NO:
            pltpu.sync_copy(table_hbm.at[pl.ds(o * N, N)], tab_v)
        else:
            pltpu.sync_copy(table_hbm, tab_v)
        zeros16 = jnp.zeros((16,), jnp.float32)

        def zbody(i, carry):
            agg_v[pl.ds(i * 16, 16)] = zeros16
            return carry

        lax.fori_loop(0, N // 16, zbody, 0, unroll=8)

        ebase = q * ES

        def fire(sv, dv, bv, sem, off):
            pltpu.async_copy(src_hbm.at[pl.ds(off, C)], sv, sem)
            pltpu.async_copy(dst_hbm.at[pl.ds(off, C)], dv, sem)
            for r in range(NO):
                @pl.when(o == r)
                def _(_r=r):
                    pltpu.async_copy(brows[_r].at[pl.ds(off, C)], bv, sem)

        def wait(sv, dv, bv, sem):
            pltpu.make_async_copy(src_hbm.at[pl.ds(0, C)], sv, sem).wait()
            pltpu.make_async_copy(dst_hbm.at[pl.ds(0, C)], dv, sem).wait()
            pltpu.make_async_copy(b0.at[pl.ds(0, C)], bv, sem).wait()

        def compute(sv, dv, bv):
            def inner(j, icarry):
                sl = pl.ds(j * 16, 16)
                si = sv[sl]
                xs = plsc.load_gather(tab_v, [si])
                msg = xs * bv[sl]
                di = dv[sl]
                plsc.addupdate_scatter(agg_v, [di], msg)
                return icarry

            lax.fori_loop(0, C // 16, inner, 0, unroll=5)

        fire(sA, dA, bA, semA, pl.multiple_of(ebase, 8))

        def pair(k, carry):
            fire(sB, dB, bB, semB,
                 pl.multiple_of(ebase + (2 * k + 1) * C, 8))
            wait(sA, dA, bA, semA)
            compute(sA, dA, bA)

            @pl.when(k < NPAIR - 1)
            def _():
                fire(sA, dA, bA, semA,
                     pl.multiple_of(ebase + (2 * k + 2) * C, 8))

            wait(sB, dB, bB, semB)
            compute(sB, dB, bB)
            return carry

        lax.fori_loop(0, NPAIR, pair, 0)
        pltpu.sync_copy(agg_v, out_hbm.at[pl.ds(wid * N, N)])

    return sc_layer


_sc_layer1 = _make_sc_layer(1)
_sc_layer2 = _make_sc_layer(NO)


def _make_sc_combine1():
    mesh = plsc.VectorSubcoreMesh(core_axis_name="c", subcore_axis_name="s")

    @functools.partial(
        pl.kernel,
        mesh=mesh,
        compiler_params=_SC_PARAMS,
        out_type=jax.ShapeDtypeStruct((NO * N,), jnp.float32),
        scratch_types=[
            pltpu.VMEM(((NQ + 1) * SEGC,), jnp.float32),
            pltpu.VMEM((SEGC,), jnp.float32),
            pltpu.VMEM((16,), jnp.float32),
            pltpu.VMEM((16,), jnp.float32),
            pltpu.SemaphoreType.DMA,
        ],
    )
    def sc_c(p_hbm, x_hbm, rb_hbm, bb_hbm, out_hbm, buf_v, out_v, r_v, b_v,
             sem):
        wid = lax.axis_index("s") * 2 + lax.axis_index("c")
        o = wid % NO
        q = wid // NO
        start = pl.multiple_of(
            jnp.minimum(q * SEGC, N - SEGC).astype(jnp.int32), 16)
        pltpu.sync_copy(rb_hbm.at[pl.ds(o * 16, 16)], r_v)
        pltpu.sync_copy(bb_hbm.at[pl.ds(o * 16, 16)], b_v)
        cps = []
        for j in range(NQ):
            cps.append(pltpu.async_copy(
                p_hbm.at[pl.ds((j * NO + o) * N + start, SEGC)],
                buf_v.at[pl.ds(j * SEGC, SEGC)], sem))
        cps.append(pltpu.async_copy(x_hbm.at[pl.ds(start, SEGC)],
                                    buf_v.at[pl.ds(NQ * SEGC, SEGC)], sem))
        for cp in cps:
            cp.wait()
        rv = r_v[...]
        bv = b_v[...]

        def fbody(k, carry):
            s = buf_v[pl.ds(k * 16, 16)]
            for j in range(1, NQ):
                s = s + buf_v[pl.ds(j * SEGC + k * 16, 16)]
            h = s + buf_v[pl.ds(NQ * SEGC + k * 16, 16)] * rv + bv
            out_v[pl.ds(k * 16, 16)] = _elu16(h)
            return carry

        lax.fori_loop(0, SEGC // 16, fbody, 0, unroll=4)
        pltpu.sync_copy(out_v, out_hbm.at[pl.ds(o * N + start, SEGC)])

    return sc_c


_sc_combine1 = _make_sc_combine1()


def _make_sc_combine2():
    mesh = plsc.VectorSubcoreMesh(core_axis_name="c", subcore_axis_name="s")

    @functools.partial(
        pl.kernel,
        mesh=mesh,
        compiler_params=_SC_PARAMS,
        out_type=jax.ShapeDtypeStruct((N,), jnp.float32),
        scratch_types=[
            pltpu.VMEM(((NT + NO) * SEGD,), jnp.float32),
            pltpu.VMEM((SEGD,), jnp.float32),
            pltpu.VMEM((128,), jnp.float32),
            pltpu.VMEM((16,), jnp.float32),
            pltpu.SemaphoreType.DMA,
        ],
    )
    def sc_d(p_hbm, h_hbm, rb_hbm, bb_hbm, out_hbm, buf_v, acc_v, r_v, b_v,
             sem):
        wid = lax.axis_index("s") * 2 + lax.axis_index("c")
        start = pl.multiple_of(
            jnp.minimum(wid * SEGD, N - SEGD).astype(jnp.int32), 16)
        pltpu.sync_copy(rb_hbm, r_v)
        pltpu.sync_copy(bb_hbm, b_v)
        cps = []
        for j in range(NT):
            cps.append(pltpu.async_copy(
                p_hbm.at[pl.ds(j * N + start, SEGD)],
                buf_v.at[pl.ds(j * SEGD, SEGD)], sem))
        for o2 in range(NO):
            cps.append(pltpu.async_copy(
                h_hbm.at[pl.ds(o2 * N + start, SEGD)],
                buf_v.at[pl.ds((NT + o2) * SEGD, SEGD)], sem))
        for cp in cps:
            cp.wait()
        rv = [r_v[pl.ds(o2 * 16, 16)] for o2 in range(NO)]
        bv = b_v[...]

        def fbody(k, carry):
            s = buf_v[pl.ds(k * 16, 16)]
            for j in range(1, NT):
                s = s + buf_v[pl.ds(j * SEGD + k * 16, 16)]
            for o2 in range(NO):
                s = s + buf_v[pl.ds((NT + o2) * SEGD + k * 16, 16)] * rv[o2]
            acc_v[pl.ds(k * 16, 16)] = _elu16(s + bv)
            return carry

        lax.fori_loop(0, SEGD // 16, fbody, 0)
        pltpu.sync_copy(acc_v, out_hbm.at[pl.ds(start, SEGD)])

    return sc_d


_sc_combine2 = _make_sc_combine2()


def kernel(x, edge_index, pseudo, W1, root1, bias1, W2, root2, bias2):
    src = edge_index[0].astype(jnp.int32)
    dst = edge_index[1].astype(jnp.int32)
    pseudo_t = jnp.pad(pseudo.T, ((0, 0), (0, E_PAD - E)))
    wcat = jnp.concatenate([W1[:, 0, :], W2[:, :, 0]], axis=1)
    mt = wcat.reshape(5, 5, 5, 16).reshape(25, 80).T
    xf = x.reshape(-1)
    rb1 = jnp.tile(root1.reshape(NO, 1), (1, 16)).reshape(-1)
    bb1 = jnp.tile(bias1.reshape(NO, 1), (1, 16)).reshape(-1)
    rb2 = jnp.tile(root2.reshape(NO, 1), (1, 16)).reshape(-1)
    bb2 = jnp.broadcast_to(bias2, (16,)).astype(jnp.float32)

    bt = _phase_a(pseudo_t, mt)
    p1 = _sc_layer1(xf, src, dst, *bt[0:8])
    htf = _sc_combine1(p1, xf, rb1, bb1)
    p2 = _sc_layer2(htf, src, dst, *bt[8:16])
    return _sc_combine2(p2, htf, rb2, bb2)


# speedup vs baseline: 87.3933x; 1.3275x over previous
import functools

import jax
import jax.numpy as jnp
from jax import lax
from jax.experimental import pallas as pl
from jax.experimental.pallas import tpu as pltpu
from jax.experimental.pallas import tpu_sc as plsc

N = 50000
E = 800000
NT = 32
NO = 8
NQ = 4
ES = E // NQ
C = 2000
NCHUNK = ES // C
NPAIR = NCHUNK // 2
E_PAD = 819200
EB = 8192

SEGC = 12544
SEGD = 1664

_SC_PARAMS = pltpu.CompilerParams(needs_layout_passes=False)


def _elu16(h):
    return jnp.where(h > 0, h, jnp.exp(jnp.minimum(h, 0.0)) - 1.0)


def _phase_a_body(pt_ref, mt_ref, *bt_refs):
    pt = pt_ref[...]
    v = pt * 4.0
    fl = jnp.floor(v)
    frac = v - fl
    k0 = jnp.clip(fl.astype(jnp.int32), 0, 3)
    io5 = lax.broadcasted_iota(jnp.int32, (5, EB), 0)

    def cdim(d):
        k = k0[d:d + 1]
        f = frac[d:d + 1]
        return jnp.where(io5 == k, 1.0 - f, jnp.where(io5 == k + 1, f, 0.0))

    c0 = cdim(0)
    c1 = cdim(1)
    c2 = cdim(2)
    c12 = (c2[:, None, :] * c1[None, :, :]).reshape(25, EB)
    dt = jax.lax.dot_general(mt_ref[...], c12, (((1,), (0,)), ((), ())),
                             preferred_element_type=jnp.float32)
    acc = dt[0:16] * c0[0:1]
    for i in range(1, 5):
        acc = acc + dt[i * 16:(i + 1) * 16] * c0[i:i + 1]
    for r in range(16):
        bt_refs[r][...] = acc[r]


def _phase_a(pseudo_t, mt):
    return pl.pallas_call(
        _phase_a_body,
        grid=(E_PAD // EB,),
        in_specs=[pl.BlockSpec((3, EB), lambda i: (0, i)),
                  pl.BlockSpec((80, 25), lambda i: (0, 0))],
        out_specs=[pl.BlockSpec((EB,), lambda i: (i,)) for _ in range(16)],
        out_shape=[jax.ShapeDtypeStruct((E_PAD,), jnp.float32)
                   for _ in range(16)],
    )(pseudo_t, mt)


def _make_sc_layer(table_rows):
    mesh = plsc.VectorSubcoreMesh(core_axis_name="c", subcore_axis_name="s")

    @functools.partial(
        pl.kernel,
        mesh=mesh,
        compiler_params=_SC_PARAMS,
        out_type=jax.ShapeDtypeStruct((NT * N,), jnp.float32),
        scratch_types=[
            pltpu.VMEM((N,), jnp.float32),
            pltpu.VMEM((N,), jnp.float32),
            pltpu.VMEM((C,), jnp.int32),
            pltpu.VMEM((C,), jnp.int32),
            pltpu.VMEM((C,), jnp.float32),
            pltpu.VMEM((C,), jnp.int32),
            pltpu.VMEM((C,), jnp.int32),
            pltpu.VMEM((C,), jnp.float32),
            pltpu.SemaphoreType.DMA,
            pltpu.SemaphoreType.DMA,
        ],
    )
    def sc_layer(table_hbm, src_hbm, dst_hbm, b0, b1, b2, b3, b4, b5, b6, b7,
                 out_hbm, tab_v, agg_v, sA, dA, bA, sB, dB, bB, semA, semB):
        brows = (b0, b1, b2, b3, b4, b5, b6, b7)
        wid = lax.axis_index("s") * 2 + lax.axis_index("c")
        o = wid % NO
        q = wid // NO
        if table_rows == NO:
            pltpu.sync_copy(table_hbm.at[pl.ds(o * N, N)], tab_v)
        else:
            pltpu.sync_copy(table_hbm, tab_v)
        zeros16 = jnp.zeros((16,), jnp.float32)

        @plsc.parallel_loop(0, N, step=16, unroll=8)
        def _zero(i):
            agg_v[pl.ds(i, 16)] = zeros16

        ebase = q * ES

        def fire(sv, dv, bv, sem, off):
            pltpu.async_copy(src_hbm.at[pl.ds(off, C)], sv, sem)
            pltpu.async_copy(dst_hbm.at[pl.ds(off, C)], dv, sem)
            for r in range(NO):
                @pl.when(o == r)
                def _(_r=r):
                    pltpu.async_copy(brows[_r].at[pl.ds(off, C)], bv, sem)

        def wait(sv, dv, bv, sem):
            pltpu.make_async_copy(src_hbm.at[pl.ds(0, C)], sv, sem).wait()
            pltpu.make_async_copy(dst_hbm.at[pl.ds(0, C)], dv, sem).wait()
            pltpu.make_async_copy(b0.at[pl.ds(0, C)], bv, sem).wait()

        def compute(sv, dv, bv):
            @plsc.parallel_loop(0, C, step=16, unroll=5)
            def _inner(j):
                sl = pl.ds(j, 16)
                si = sv[sl]
                xs = plsc.load_gather(tab_v, [si])
                msg = xs * bv[sl]
                di = dv[sl]
                plsc.addupdate_scatter(agg_v, [di], msg)

        fire(sA, dA, bA, semA, pl.multiple_of(ebase, 8))

        def pair(k, carry):
            fire(sB, dB, bB, semB,
                 pl.multiple_of(ebase + (2 * k + 1) * C, 8))
            wait(sA, dA, bA, semA)
            compute(sA, dA, bA)

            @pl.when(k < NPAIR - 1)
            def _():
                fire(sA, dA, bA, semA,
                     pl.multiple_of(ebase + (2 * k + 2) * C, 8))

            wait(sB, dB, bB, semB)
            compute(sB, dB, bB)
            return carry

        lax.fori_loop(0, NPAIR, pair, 0)
        pltpu.sync_copy(agg_v, out_hbm.at[pl.ds(wid * N, N)])

    return sc_layer


_sc_layer1 = _make_sc_layer(1)
_sc_layer2 = _make_sc_layer(NO)


def _make_sc_combine1():
    mesh = plsc.VectorSubcoreMesh(core_axis_name="c", subcore_axis_name="s")

    @functools.partial(
        pl.kernel,
        mesh=mesh,
        compiler_params=_SC_PARAMS,
        out_type=jax.ShapeDtypeStruct((NO * N,), jnp.float32),
        scratch_types=[
            pltpu.VMEM(((NQ + 1) * SEGC,), jnp.float32),
            pltpu.VMEM((SEGC,), jnp.float32),
            pltpu.VMEM((16,), jnp.float32),
            pltpu.VMEM((16,), jnp.float32),
            pltpu.SemaphoreType.DMA,
        ],
    )
    def sc_c(p_hbm, x_hbm, rb_hbm, bb_hbm, out_hbm, buf_v, out_v, r_v, b_v,
             sem):
        wid = lax.axis_index("s") * 2 + lax.axis_index("c")
        o = wid % NO
        q = wid // NO
        start = pl.multiple_of(
            jnp.minimum(q * SEGC, N - SEGC).astype(jnp.int32), 16)
        pltpu.sync_copy(rb_hbm.at[pl.ds(o * 16, 16)], r_v)
        pltpu.sync_copy(bb_hbm.at[pl.ds(o * 16, 16)], b_v)
        cps = []
        for j in range(NQ):
            cps.append(pltpu.async_copy(
                p_hbm.at[pl.ds((j * NO + o) * N + start, SEGC)],
                buf_v.at[pl.ds(j * SEGC, SEGC)], sem))
        cps.append(pltpu.async_copy(x_hbm.at[pl.ds(start, SEGC)],
                                    buf_v.at[pl.ds(NQ * SEGC, SEGC)], sem))
        for cp in cps:
            cp.wait()
        rv = r_v[...]
        bv = b_v[...]

        def fbody(k, carry):
            s = buf_v[pl.ds(k * 16, 16)]
            for j in range(1, NQ):
                s = s + buf_v[pl.ds(j * SEGC + k * 16, 16)]
            h = s + buf_v[pl.ds(NQ * SEGC + k * 16, 16)] * rv + bv
            out_v[pl.ds(k * 16, 16)] = _elu16(h)
            return carry

        lax.fori_loop(0, SEGC // 16, fbody, 0, unroll=4)
        pltpu.sync_copy(out_v, out_hbm.at[pl.ds(o * N + start, SEGC)])

    return sc_c


_sc_combine1 = _make_sc_combine1()


def _make_sc_combine2():
    mesh = plsc.VectorSubcoreMesh(core_axis_name="c", subcore_axis_name="s")

    @functools.partial(
        pl.kernel,
        mesh=mesh,
        compiler_params=_SC_PARAMS,
        out_type=jax.ShapeDtypeStruct((N,), jnp.float32),
        scratch_types=[
            pltpu.VMEM(((NT + NO) * SEGD,), jnp.float32),
            pltpu.VMEM((SEGD,), jnp.float32),
            pltpu.VMEM((128,), jnp.float32),
            pltpu.VMEM((16,), jnp.float32),
            pltpu.SemaphoreType.DMA,
        ],
    )
    def sc_d(p_hbm, h_hbm, rb_hbm, bb_hbm, out_hbm, buf_v, acc_v, r_v, b_v,
             sem):
        wid = lax.axis_index("s") * 2 + lax.axis_index("c")
        start = pl.multiple_of(
            jnp.minimum(wid * SEGD, N - SEGD).astype(jnp.int32), 16)
        pltpu.sync_copy(rb_hbm, r_v)
        pltpu.sync_copy(bb_hbm, b_v)
        cps = []
        for j in range(NT):
            cps.append(pltpu.async_copy(
                p_hbm.at[pl.ds(j * N + start, SEGD)],
                buf_v.at[pl.ds(j * SEGD, SEGD)], sem))
        for o2 in range(NO):
            cps.append(pltpu.async_copy(
                h_hbm.at[pl.ds(o2 * N + start, SEGD)],
                buf_v.at[pl.ds((NT + o2) * SEGD, SEGD)], sem))
        for cp in cps:
            cp.wait()
        rv = [r_v[pl.ds(o2 * 16, 16)] for o2 in range(NO)]
        bv = b_v[...]

        def fbody(k, carry):
            s = buf_v[pl.ds(k * 16, 16)]
            for j in range(1, NT):
                s = s + buf_v[pl.ds(j * SEGD + k * 16, 16)]
            for o2 in range(NO):
                s = s + buf_v[pl.ds((NT + o2) * SEGD + k * 16, 16)] * rv[o2]
            acc_v[pl.ds(k * 16, 16)] = _elu16(s + bv)
            return carry

        lax.fori_loop(0, SEGD // 16, fbody, 0)
        pltpu.sync_copy(acc_v, out_hbm.at[pl.ds(start, SEGD)])

    return sc_d


_sc_combine2 = _make_sc_combine2()


def kernel(x, edge_index, pseudo, W1, root1, bias1, W2, root2, bias2):
    src = edge_index[0].astype(jnp.int32)
    dst = edge_index[1].astype(jnp.int32)
    pseudo_t = jnp.pad(pseudo.T, ((0, 0), (0, E_PAD - E)))
    wcat = jnp.concatenate([W1[:, 0, :], W2[:, :, 0]], axis=1)
    mt = wcat.reshape(5, 5, 5, 16).reshape(25, 80).T
    xf = x.reshape(-1)
    rb1 = jnp.tile(root1.reshape(NO, 1), (1, 16)).reshape(-1)
    bb1 = jnp.tile(bias1.reshape(NO, 1), (1, 16)).reshape(-1)
    rb2 = jnp.tile(root2.reshape(NO, 1), (1, 16)).reshape(-1)
    bb2 = jnp.broadcast_to(bias2, (16,)).astype(jnp.float32)

    bt = _phase_a(pseudo_t, mt)
    p1 = _sc_layer1(xf, src, dst, *bt[0:8])
    htf = _sc_combine1(p1, xf, rb1, bb1)
    p2 = _sc_layer2(htf, src, dst, *bt[8:16])
    return _sc_combine2(p2, htf, rb2, bb2)


# speedup vs baseline: 95.9616x; 1.0980x over previous
import functools

import jax
import jax.numpy as jnp
from jax import lax
from jax.experimental import pallas as pl
from jax.experimental.pallas import tpu as pltpu
from jax.experimental.pallas import tpu_sc as plsc

N = 50000
E = 800000
NT = 32
NO = 8
NQ = 4
ES = E // NQ
C = 2000
NCHUNK = ES // C
NPAIR = NCHUNK // 2
E_PAD = 819200
EB = 8192

SEGC = 12544
SEGD = 1664

_SC_PARAMS = pltpu.CompilerParams(needs_layout_passes=False)


def _elu16(h):
    return jnp.where(h > 0, h, jnp.exp(jnp.minimum(h, 0.0)) - 1.0)


def _phase_a_body(pt_ref, mt_ref, *bt_refs):
    pt = pt_ref[...]
    v = pt * 4.0
    io8 = lax.broadcasted_iota(jnp.int32, (8, EB), 0).astype(jnp.float32)

    def cdim(d):
        return jnp.maximum(1.0 - jnp.abs(io8 - v[d:d + 1]), 0.0)

    c0 = cdim(0)
    c1 = cdim(1)
    c2 = cdim(2)
    c12 = (c2[:, None, :] * c1[None, :, :]).reshape(64, EB)
    dt = jax.lax.dot_general(mt_ref[...], c12, (((1,), (0,)), ((), ())),
                             preferred_element_type=jnp.float32)
    acc = dt[0:16] * c0[0:1]
    for i in range(1, 5):
        acc = acc + dt[i * 16:(i + 1) * 16] * c0[i:i + 1]
    for r in range(16):
        bt_refs[r][...] = acc[r]


def _phase_a(pseudo_t, mt):
    return pl.pallas_call(
        _phase_a_body,
        grid=(E_PAD // EB,),
        in_specs=[pl.BlockSpec((3, EB), lambda i: (0, i)),
                  pl.BlockSpec((80, 64), lambda i: (0, 0))],
        out_specs=[pl.BlockSpec((EB,), lambda i: (i,)) for _ in range(16)],
        out_shape=[jax.ShapeDtypeStruct((E_PAD,), jnp.float32)
                   for _ in range(16)],
    )(pseudo_t, mt)


def _make_sc_layer(table_rows):
    mesh = plsc.VectorSubcoreMesh(core_axis_name="c", subcore_axis_name="s")

    @functools.partial(
        pl.kernel,
        mesh=mesh,
        compiler_params=_SC_PARAMS,
        out_type=jax.ShapeDtypeStruct((NT * N,), jnp.float32),
        scratch_types=[
            pltpu.VMEM((N,), jnp.float32),
            pltpu.VMEM((N,), jnp.float32),
            pltpu.VMEM((C,), jnp.int32),
            pltpu.VMEM((C,), jnp.int32),
            pltpu.VMEM((C,), jnp.float32),
            pltpu.VMEM((C,), jnp.int32),
            pltpu.VMEM((C,), jnp.int32),
            pltpu.VMEM((C,), jnp.float32),
            pltpu.SemaphoreType.DMA,
            pltpu.SemaphoreType.DMA,
        ],
    )
    def sc_layer(table_hbm, src_hbm, dst_hbm, b0, b1, b2, b3, b4, b5, b6, b7,
                 out_hbm, tab_v, agg_v, sA, dA, bA, sB, dB, bB, semA, semB):
        brows = (b0, b1, b2, b3, b4, b5, b6, b7)
        wid = lax.axis_index("s") * 2 + lax.axis_index("c")
        o = wid % NO
        q = wid // NO
        if table_rows == NO:
            pltpu.sync_copy(table_hbm.at[pl.ds(o * N, N)], tab_v)
        else:
            pltpu.sync_copy(table_hbm, tab_v)
        zeros16 = jnp.zeros((16,), jnp.float32)

        @plsc.parallel_loop(0, N, step=16, unroll=8)
        def _zero(i):
            agg_v[pl.ds(i, 16)] = zeros16

        ebase = q * ES

        def fire(sv, dv, bv, sem, off):
            pltpu.async_copy(src_hbm.at[pl.ds(off, C)], sv, sem)
            pltpu.async_copy(dst_hbm.at[pl.ds(off, C)], dv, sem)
            for r in range(NO):
                @pl.when(o == r)
                def _(_r=r):
                    pltpu.async_copy(brows[_r].at[pl.ds(off, C)], bv, sem)

        def wait(sv, dv, bv, sem):
            pltpu.make_async_copy(src_hbm.at[pl.ds(0, C)], sv, sem).wait()
            pltpu.make_async_copy(dst_hbm.at[pl.ds(0, C)], dv, sem).wait()
            pltpu.make_async_copy(b0.at[pl.ds(0, C)], bv, sem).wait()

        def compute(sv, dv, bv):
            @plsc.parallel_loop(0, C, step=16, unroll=5)
            def _inner(j):
                sl = pl.ds(j, 16)
                si = sv[sl]
                xs = plsc.load_gather(tab_v, [si])
                msg = xs * bv[sl]
                di = dv[sl]
                plsc.addupdate_scatter(agg_v, [di], msg)

        fire(sA, dA, bA, semA, pl.multiple_of(ebase, 8))

        def pair(k, carry):
            fire(sB, dB, bB, semB,
                 pl.multiple_of(ebase + (2 * k + 1) * C, 8))
            wait(sA, dA, bA, semA)
            compute(sA, dA, bA)

            @pl.when(k < NPAIR - 1)
            def _():
                fire(sA, dA, bA, semA,
                     pl.multiple_of(ebase + (2 * k + 2) * C, 8))

            wait(sB, dB, bB, semB)
            compute(sB, dB, bB)
            return carry

        lax.fori_loop(0, NPAIR, pair, 0)
        pltpu.sync_copy(agg_v, out_hbm.at[pl.ds(wid * N, N)])

    return sc_layer


_sc_layer1 = _make_sc_layer(1)
_sc_layer2 = _make_sc_layer(NO)


def _make_sc_combine1():
    mesh = plsc.VectorSubcoreMesh(core_axis_name="c", subcore_axis_name="s")

    @functools.partial(
        pl.kernel,
        mesh=mesh,
        compiler_params=_SC_PARAMS,
        out_type=jax.ShapeDtypeStruct((NO * N,), jnp.float32),
        scratch_types=[
            pltpu.VMEM(((NQ + 1) * SEGC,), jnp.float32),
            pltpu.VMEM((SEGC,), jnp.float32),
            pltpu.VMEM((16,), jnp.float32),
            pltpu.VMEM((16,), jnp.float32),
            pltpu.SemaphoreType.DMA,
        ],
    )
    def sc_c(p_hbm, x_hbm, rb_hbm, bb_hbm, out_hbm, buf_v, out_v, r_v, b_v,
             sem):
        wid = lax.axis_index("s") * 2 + lax.axis_index("c")
        o = wid % NO
        q = wid // NO
        start = pl.multiple_of(
            jnp.minimum(q * SEGC, N - SEGC).astype(jnp.int32), 16)
        pltpu.sync_copy(rb_hbm.at[pl.ds(o * 16, 16)], r_v)
        pltpu.sync_copy(bb_hbm.at[pl.ds(o * 16, 16)], b_v)
        cps = []
        for j in range(NQ):
            cps.append(pltpu.async_copy(
                p_hbm.at[pl.ds((j * NO + o) * N + start, SEGC)],
                buf_v.at[pl.ds(j * SEGC, SEGC)], sem))
        cps.append(pltpu.async_copy(x_hbm.at[pl.ds(start, SEGC)],
                                    buf_v.at[pl.ds(NQ * SEGC, SEGC)], sem))
        for cp in cps:
            cp.wait()
        rv = r_v[...]
        bv = b_v[...]

        def fbody(k, carry):
            s = buf_v[pl.ds(k * 16, 16)]
            for j in range(1, NQ):
                s = s + buf_v[pl.ds(j * SEGC + k * 16, 16)]
            h = s + buf_v[pl.ds(NQ * SEGC + k * 16, 16)] * rv + bv
            out_v[pl.ds(k * 16, 16)] = _elu16(h)
            return carry

        lax.fori_loop(0, SEGC // 16, fbody, 0, unroll=4)
        pltpu.sync_copy(out_v, out_hbm.at[pl.ds(o * N + start, SEGC)])

    return sc_c


_sc_combine1 = _make_sc_combine1()


def _make_sc_combine2():
    mesh = plsc.VectorSubcoreMesh(core_axis_name="c", subcore_axis_name="s")

    @functools.partial(
        pl.kernel,
        mesh=mesh,
        compiler_params=_SC_PARAMS,
        out_type=jax.ShapeDtypeStruct((N,), jnp.float32),
        scratch_types=[
            pltpu.VMEM(((NT + NO) * SEGD,), jnp.float32),
            pltpu.VMEM((SEGD,), jnp.float32),
            pltpu.VMEM((128,), jnp.float32),
            pltpu.VMEM((16,), jnp.float32),
            pltpu.SemaphoreType.DMA,
        ],
    )
    def sc_d(p_hbm, h_hbm, rb_hbm, bb_hbm, out_hbm, buf_v, acc_v, r_v, b_v,
             sem):
        wid = lax.axis_index("s") * 2 + lax.axis_index("c")
        start = pl.multiple_of(
            jnp.minimum(wid * SEGD, N - SEGD).astype(jnp.int32), 16)
        pltpu.sync_copy(rb_hbm, r_v)
        pltpu.sync_copy(bb_hbm, b_v)
        cps = []
        for j in range(NT):
            cps.append(pltpu.async_copy(
                p_hbm.at[pl.ds(j * N + start, SEGD)],
                buf_v.at[pl.ds(j * SEGD, SEGD)], sem))
        for o2 in range(NO):
            cps.append(pltpu.async_copy(
                h_hbm.at[pl.ds(o2 * N + start, SEGD)],
                buf_v.at[pl.ds((NT + o2) * SEGD, SEGD)], sem))
        for cp in cps:
            cp.wait()
        rv = [r_v[pl.ds(o2 * 16, 16)] for o2 in range(NO)]
        bv = b_v[...]

        def fbody(k, carry):
            s = buf_v[pl.ds(k * 16, 16)]
            for j in range(1, NT):
                s = s + buf_v[pl.ds(j * SEGD + k * 16, 16)]
            for o2 in range(NO):
                s = s + buf_v[pl.ds((NT + o2) * SEGD + k * 16, 16)] * rv[o2]
            acc_v[pl.ds(k * 16, 16)] = _elu16(s + bv)
            return carry

        lax.fori_loop(0, SEGD // 16, fbody, 0)
        pltpu.sync_copy(acc_v, out_hbm.at[pl.ds(start, SEGD)])

    return sc_d


_sc_combine2 = _make_sc_combine2()


def kernel(x, edge_index, pseudo, W1, root1, bias1, W2, root2, bias2):
    src = edge_index[0].astype(jnp.int32)
    dst = edge_index[1].astype(jnp.int32)
    pseudo_t = jnp.pad(pseudo.T, ((0, 0), (0, E_PAD - E)))
    wcat = jnp.concatenate([W1[:, 0, :], W2[:, :, 0]], axis=1)
    mt25 = wcat.reshape(5, 5, 5, 16).reshape(25, 80).T
    cols = jnp.array([8 * (m // 5) + m % 5 for m in range(25)], jnp.int32)
    mt = jnp.zeros((80, 64), jnp.float32).at[:, cols].set(mt25)
    xf = x.reshape(-1)
    rb1 = jnp.tile(root1.reshape(NO, 1), (1, 16)).reshape(-1)
    bb1 = jnp.tile(bias1.reshape(NO, 1), (1, 16)).reshape(-1)
    rb2 = jnp.tile(root2.reshape(NO, 1), (1, 16)).reshape(-1)
    bb2 = jnp.broadcast_to(bias2, (16,)).astype(jnp.float32)

    bt = _phase_a(pseudo_t, mt)
    p1 = _sc_layer1(xf, src, dst, *bt[0:8])
    htf = _sc_combine1(p1, xf, rb1, bb1)
    p2 = _sc_layer2(htf, src, dst, *bt[8:16])
    return _sc_combine2(p2, htf, rb2, bb2)
